# baseline (speedup 1.0000x reference)
import functools

import jax
import jax.numpy as jnp
from jax import lax
from jax.experimental import pallas as pl
from jax.experimental.pallas import tpu as pltpu
from jax.experimental.pallas import tpu_sc as plsc

N = 10000
E = 320000
D = 128
H = 512
B = 16

NC = 2
NS = 16
CW = 128
NCHUNK = H // CW
EB = 128
NB = E // EB
NP = 10240
RPT = NP // NS

F32 = jnp.float32
HIGHEST = lax.Precision.DEFAULT

@functools.cache
def _sc_mesh():
    return plsc.VectorSubcoreMesh(core_axis_name="c", subcore_axis_name="s",
                                  num_cores=NC, num_subcores=NS)


def _hist_body(col2d, cnt_out, cnt_sh, colb, onesb, zb):
    core = lax.axis_index("c")
    tid = lax.axis_index("s")
    wid = core * NS + tid
    z16 = jnp.zeros((16,), F32)

    def zfill(i, carry):
        zb[pl.ds(i * 16, 16)] = z16
        return carry

    lax.fori_loop(0, 640 // 16, zfill, None)
    for j in range(EB // 16):
        onesb[pl.ds(j * 16, 16)] = jnp.ones((16,), F32)

    pltpu.sync_copy(zb, cnt_sh.at[pl.ds(tid * 640, 640)])
    plsc.subcore_barrier()

    lo = wid * NB // (NC * NS)
    hi = (wid + 1) * NB // (NC * NS)

    def body(j, carry):
        pltpu.sync_copy(col2d.at[j], colb)
        pltpu.sync_copy(onesb, cnt_sh.at[colb], add=True)
        return carry

    lax.fori_loop(lo, hi, body, None)
    plsc.subcore_barrier()

    pltpu.sync_copy(cnt_sh.at[pl.ds(tid * 640, 640)], zb)
    pltpu.sync_copy(zb, cnt_out.at[core, pl.ds(tid * 640, 640)])


def _hist(col2d):
    k = pl.kernel(
        _hist_body,
        out_type=jax.ShapeDtypeStruct((NC, NP), F32),
        mesh=_sc_mesh(),
        scratch_types=[
            pltpu.VMEM_SHARED((NP,), F32),
            pltpu.VMEM((EB,), jnp.int32),
            pltpu.VMEM((EB,), F32),
            pltpu.VMEM((640,), F32),
        ],
    )
    return k(col2d)


RPB = 64


def _agg_body(rc2d, hs4, agg_out, acc, rcg, gbuf, bb,
              semg0, semg1, semsc0, semsc1, semi0, semi1, semi2, semi3):
    core = lax.axis_index("c")
    tid = lax.axis_index("s")
    glo = tid * NB // NS
    ghi = (tid + 1) * NB // NS
    z16 = jnp.zeros((16,), F32)
    semg = (semg0, semg1)
    semsc = (semsc0, semsc1)
    semi = (semi0, semi1, semi2, semi3)

    def edge_loop(chunk):
        hs_c = hs4.at[chunk]

        def stage(g, s4):
            pltpu.async_copy(rc2d.at[g], rcg.at[s4], semi[s4])

        def stage_wait(s4):
            pltpu.make_async_copy(rc2d.at[0], rcg.at[s4], semi[s4]).wait()

        def gather_fire(s4, p):
            stage_wait(s4)
            pltpu.async_copy(hs_c.at[rcg.at[s4, 0]], gbuf.at[p], semg[p])

        def gather_wait(p):
            pltpu.make_async_copy(hs_c.at[pl.ds(0, EB), :],
                                  gbuf.at[p], semg[p]).wait()

        def scatter_fire(p, s4):
            pltpu.async_copy(gbuf.at[p], acc.at[rcg.at[s4, 1]], semsc[p],
                             add=True)

        def scatter_wait(p):
            pltpu.make_async_copy(hs_c.at[pl.ds(0, EB), :],
                                  gbuf.at[p], semsc[p]).wait()

        def full_iter(g, k):
            p = k % 2
            gather_wait(p ^ 1)
            scatter_fire(p ^ 1, (k - 1) % 4)
            scatter_wait(p)
            gather_fire(k, p)
            stage(jnp.minimum(g + 2, ghi - 1), (k + 2) % 4)

        for k0 in range(4):
            @pl.when(glo % 4 == k0)
            def _(k0=k0):
                k1 = (k0 + 1) % 4
                stage(glo, k0)
                stage(glo + 1, k1)
                gather_fire(k0, k0 % 2)
                stage(jnp.minimum(glo + 2, ghi - 1), (k0 + 2) % 4)
                gather_fire(k1, k1 % 2)
                gather_wait(k0 % 2)
                scatter_fire(k0 % 2, k0)
                stage(jnp.minimum(glo + 3, ghi - 1), (k1 + 2) % 4)

        def body(g, carry):
            for k in range(4):
                @pl.when(g % 4 == k)
                def _(k=k):
                    full_iter(g, k)
            return carry

        lax.fori_loop(glo + 2, ghi, body, None)

        for kl in range(4):
            @pl.when((ghi - 1) % 4 == kl)
            def _(kl=kl):
                pl_ = kl % 2
                gather_wait(pl_)
                scatter_fire(pl_, kl)
                scatter_wait(pl_ ^ 1)
                scatter_wait(pl_)
                stage_wait((kl + 1) % 4)
                stage_wait((kl + 2) % 4)

    def writeback(chunk):
        for k in range(RPT // RPB):
            off = tid * RPT + k * RPB
            pltpu.sync_copy(acc.at[pl.ds(off, RPB), :], bb)
            pltpu.sync_copy(bb, agg_out.at[chunk, pl.ds(off, RPB), :])

    for p in range(2):
        def zfill(i, carry):
            for j in range(CW // 16):
                bb[i, pl.ds(j * 16, 16)] = z16
            return carry

        lax.fori_loop(0, RPB, zfill, None)
        for k in range(RPT // RPB):
            pltpu.sync_copy(bb, acc.at[pl.ds(tid * RPT + k * RPB, RPB), :])
        plsc.subcore_barrier()

        @pl.when(core == 0)
        def _():
            edge_loop(2 * p)

        @pl.when(core == 1)
        def _():
            edge_loop(2 * p + 1)

        plsc.subcore_barrier()

        @pl.when(core == 0)
        def _():
            writeback(2 * p)

        @pl.when(core == 1)
        def _():
            writeback(2 * p + 1)

        plsc.subcore_barrier()


def _agg(rc2d, hs4):
    k = pl.kernel(
        _agg_body,
        out_type=jax.ShapeDtypeStruct((NCHUNK, NP, CW), F32),
        mesh=_sc_mesh(),
        scratch_types=[
            pltpu.VMEM_SHARED((NP, CW), F32),
            pltpu.VMEM((4, 2, EB), jnp.int32),
            pltpu.VMEM((2, EB, CW), F32),
            pltpu.VMEM((RPB, CW), F32),
            pltpu.SemaphoreType.DMA,
            pltpu.SemaphoreType.DMA,
            pltpu.SemaphoreType.DMA,
            pltpu.SemaphoreType.DMA,
            pltpu.SemaphoreType.DMA,
            pltpu.SemaphoreType.DMA,
            pltpu.SemaphoreType.DMA,
            pltpu.SemaphoreType.DMA,
        ],
    )
    return k(rc2d, hs4)


RB = 1000
GRID = N // RB


def _prep_body(cnt_ref, dis_ref):
    c = cnt_ref[...]
    ones = jnp.ones((NC, 1), F32)
    deg = lax.dot_general(c, ones, (((0,), (0,)), ((), ())),
                          precision=HIGHEST) + 1.0
    dis_ref[...] = lax.rsqrt(deg)[:N]


def _prep(cnt):
    return pl.pallas_call(
        _prep_body,
        out_shape=jax.ShapeDtypeStruct((N, 1), F32),
    )(cnt)


def _dense1_body(x_ref, w_ref, dis_ref, hs_ref):
    y = lax.dot_general(x_ref[...], w_ref[...], (((1,), (0,)), ((), ())),
                        precision=HIGHEST)
    dis = dis_ref[...]
    for c in range(NCHUNK):
        hs_ref[c] = y[:, c * CW:(c + 1) * CW] * dis


def _dense1(x, w1, dis2d):
    return pl.pallas_call(
        _dense1_body,
        grid=(GRID,),
        in_specs=[
            pl.BlockSpec((RB, D), lambda i: (i, 0)),
            pl.BlockSpec((D, H), lambda i: (0, 0)),
            pl.BlockSpec((RB, 1), lambda i: (i, 0)),
        ],
        out_specs=pl.BlockSpec((NCHUNK, RB, CW), lambda i: (0, i, 0)),
        out_shape=jax.ShapeDtypeStruct((NCHUNK, N, CW), F32),
    )(x, w1, dis2d)


def _leaky(t):
    return jnp.where(t >= 0, t, 0.01 * t)


def _combine_h(agg_ref, hs_ref, dis_ref, b_ref):
    dis = dis_ref[...]
    bfull = b_ref[...]
    parts = []
    for c in range(NCHUNK):
        t = dis * (agg_ref[c] + hs_ref[c]) + bfull[c * CW:(c + 1) * CW][None, :]
        parts.append(_leaky(t))
    return jnp.concatenate(parts, axis=1)


def _dense2_body(agg_ref, hs_ref, dis_ref, b_ref, w_ref, out_ref):
    h = _combine_h(agg_ref, hs_ref, dis_ref, b_ref)
    y = lax.dot_general(h, w_ref[...], (((1,), (0,)), ((), ())),
                        precision=HIGHEST)
    dis = dis_ref[...]
    for c in range(NCHUNK):
        out_ref[c] = y[:, c * CW:(c + 1) * CW] * dis


def _dense2(agg, hs, dis2d, b_prev, w):
    return pl.pallas_call(
        _dense2_body,
        grid=(GRID,),
        in_specs=[
            pl.BlockSpec((NCHUNK, RB, CW), lambda i: (0, i, 0)),
            pl.BlockSpec((NCHUNK, RB, CW), lambda i: (0, i, 0)),
            pl.BlockSpec((RB, 1), lambda i: (i, 0)),
            pl.BlockSpec((H,), lambda i: (0,)),
            pl.BlockSpec((H, H), lambda i: (0, 0)),
        ],
        out_specs=pl.BlockSpec((NCHUNK, RB, CW), lambda i: (0, i, 0)),
        out_shape=jax.ShapeDtypeStruct((NCHUNK, N, CW), F32),
    )(agg, hs, dis2d, b_prev, w)


def _attn_a_body(agg_ref, hs_ref, dis_ref, b_ref, wv_ref, bv_ref, wu_ref,
                 bu_ref, wa_ref, ba_ref, h_ref, a_ref):
    h = _combine_h(agg_ref, hs_ref, dis_ref, b_ref)
    h_ref[...] = h
    av = jnp.tanh(lax.dot_general(h, wv_ref[...], (((1,), (0,)), ((), ())),
                                  precision=HIGHEST) + bv_ref[...][None, :])
    u = lax.dot_general(h, wu_ref[...], (((1,), (0,)), ((), ())),
                        precision=HIGHEST) + bu_ref[...][None, :]
    au = 1.0 / (1.0 + jnp.exp(-u))
    g = av * au
    a_ref[...] = lax.dot_general(g, wa_ref[...], (((1,), (0,)), ((), ())),
                                 precision=HIGHEST) + ba_ref[0, 0]


def _attn_a(agg, hs, dis2d, b3, wv, bv, wu, bu, wa, ba2):
    return pl.pallas_call(
        _attn_a_body,
        grid=(GRID,),
        in_specs=[
            pl.BlockSpec((NCHUNK, RB, CW), lambda i: (0, i, 0)),
            pl.BlockSpec((NCHUNK, RB, CW), lambda i: (0, i, 0)),
            pl.BlockSpec((RB, 1), lambda i: (i, 0)),
            pl.BlockSpec((H,), lambda i: (0,)),
            pl.BlockSpec((H, CW), lambda i: (0, 0)),
            pl.BlockSpec((CW,), lambda i: (0,)),
            pl.BlockSpec((H, CW), lambda i: (0, 0)),
            pl.BlockSpec((CW,), lambda i: (0,)),
            pl.BlockSpec((CW, 1), lambda i: (0, 0)),
            pl.BlockSpec((1, 1), lambda i: (0, 0)),
        ],
        out_specs=[
            pl.BlockSpec((RB, H), lambda i: (i, 0)),
            pl.BlockSpec((RB, 1), lambda i: (i, 0)),
        ],
        out_shape=[
            jax.ShapeDtypeStruct((N, H), F32),
            jax.ShapeDtypeStruct((N, 1), F32),
        ],
    )(agg, hs, dis2d, b3, wv, bv, wu, bu, wa, ba2)


def _pool_body(h_ref, a_in_ref, batch_ref, wc_ref, bc_ref, a_ref, out_ref):
    bidx = batch_ref[...]
    lanes = lax.broadcasted_iota(jnp.int32, (1, B), 1)
    maskb = bidx == lanes
    mask = maskb.astype(F32)
    av = a_in_ref[...]
    neg = jnp.float32(-1e30)
    m_seg = jnp.max(jnp.where(maskb, av, neg), axis=0, keepdims=True)
    m_node = jnp.max(jnp.where(maskb, m_seg, neg), axis=1, keepdims=True)
    e = jnp.exp(av - m_node)
    s_seg = jnp.sum(mask * e, axis=0, keepdims=True)
    s_node = jnp.sum(mask * s_seg, axis=1, keepdims=True)
    a = e / (s_node + 1e-16)
    a_ref[...] = a
    w = mask * a
    z = lax.dot_general(w, h_ref[...], (((0,), (0,)), ((), ())),
                        precision=lax.Precision.HIGHEST)
    out_ref[...] = lax.dot_general(z, wc_ref[...], (((1,), (0,)), ((), ())),
                                   precision=HIGHEST) + bc_ref[0, 0]


def _pool(h3, a_node, batch2d, wc, bc2):
    return pl.pallas_call(
        _pool_body,
        out_shape=[
            jax.ShapeDtypeStruct((N, 1), F32),
            jax.ShapeDtypeStruct((B, 1), F32),
        ],
    )(h3, a_node, batch2d, wc, bc2)


def kernel(x, edge_index, batch, W1, b1, W2, b2, W3, b3, WV, bV, WU, bU,
           Wa, ba, Wc, bc):
    row2d = edge_index[0].reshape(NB, EB)
    col2d = edge_index[1].reshape(NB, EB)
    rc2d = jnp.stack([row2d, col2d], axis=1)
    batch2d = batch.reshape(N, 1)
    ba2 = ba.reshape(1, 1)
    bc2 = bc.reshape(1, 1)

    cnt = _hist(col2d)
    dis2d = _prep(cnt)

    hs1 = _dense1(x, W1, dis2d)
    agg1 = _agg(rc2d, hs1)
    hs2 = _dense2(agg1, hs1, dis2d, b1, W2)
    agg2 = _agg(rc2d, hs2)
    hs3 = _dense2(agg2, hs2, dis2d, b2, W3)
    agg3 = _agg(rc2d, hs3)
    h3, a_node = _attn_a(agg3, hs3, dis2d, b3, WV, bV, WU, bU, Wa, ba2)
    a2d, outv = _pool(h3, a_node, batch2d, Wc, bc2)

    return outv.reshape(B), a2d.reshape(N), h3


# speedup vs baseline: 1.2101x; 1.2101x over previous
import functools

import jax
import jax.numpy as jnp
from jax import lax
from jax.experimental import pallas as pl
from jax.experimental.pallas import tpu as pltpu
from jax.experimental.pallas import tpu_sc as plsc

N = 10000
E = 320000
D = 128
H = 512
B = 16

NC = 2
NS = 16
CW = 128
NCHUNK = H // CW
EB = 128
NB = E // EB
NP = 10240
RPT = NP // NS

F32 = jnp.float32
HIGHEST = lax.Precision.DEFAULT

@functools.cache
def _sc_mesh():
    return plsc.VectorSubcoreMesh(core_axis_name="c", subcore_axis_name="s",
                                  num_cores=NC, num_subcores=NS)


def _hist_body(col2d, cnt_out, cnt_sh, colb, onesb, zb):
    core = lax.axis_index("c")
    tid = lax.axis_index("s")
    wid = core * NS + tid
    z16 = jnp.zeros((16,), F32)

    def zfill(i, carry):
        zb[pl.ds(i * 16, 16)] = z16
        return carry

    lax.fori_loop(0, 640 // 16, zfill, None)
    for j in range(EB // 16):
        onesb[pl.ds(j * 16, 16)] = jnp.ones((16,), F32)

    pltpu.sync_copy(zb, cnt_sh.at[pl.ds(tid * 640, 640)])
    plsc.subcore_barrier()

    lo = wid * NB // (NC * NS)
    hi = (wid + 1) * NB // (NC * NS)

    def body(j, carry):
        pltpu.sync_copy(col2d.at[j], colb)
        pltpu.sync_copy(onesb, cnt_sh.at[colb], add=True)
        return carry

    lax.fori_loop(lo, hi, body, None)
    plsc.subcore_barrier()

    pltpu.sync_copy(cnt_sh.at[pl.ds(tid * 640, 640)], zb)
    pltpu.sync_copy(zb, cnt_out.at[core, pl.ds(tid * 640, 640)])


def _hist(col2d):
    k = pl.kernel(
        _hist_body,
        out_type=jax.ShapeDtypeStruct((NC, NP), F32),
        mesh=_sc_mesh(),
        scratch_types=[
            pltpu.VMEM_SHARED((NP,), F32),
            pltpu.VMEM((EB,), jnp.int32),
            pltpu.VMEM((EB,), F32),
            pltpu.VMEM((640,), F32),
        ],
    )
    return k(col2d)


RPB = 64


def _agg_body(row2d, col2d, hs4, agg_out, acc, rowg, colg, gbuf, bb,
              semg0, semg1, semsc0, semsc1, semi0, semi1, semi2, semi3):
    core = lax.axis_index("c")
    tid = lax.axis_index("s")
    glo = tid * NB // NS
    ghi = (tid + 1) * NB // NS
    z16 = jnp.zeros((16,), F32)
    semg = (semg0, semg1)
    semsc = (semsc0, semsc1)
    semi = (semi0, semi1, semi2, semi3)

    def edge_loop(chunk):
        hs_c = hs4.at[chunk]

        def stage(g, s4):
            pltpu.async_copy(row2d.at[g], rowg.at[s4], semi[s4])
            pltpu.async_copy(col2d.at[g], colg.at[s4], semi[s4])

        def stage_wait(s4):
            pltpu.make_async_copy(row2d.at[0], rowg.at[s4], semi[s4]).wait()
            pltpu.make_async_copy(col2d.at[0], colg.at[s4], semi[s4]).wait()

        def gather_fire(s4, p):
            stage_wait(s4)
            pltpu.async_copy(hs_c.at[rowg.at[s4]], gbuf.at[p], semg[p])

        def gather_wait(p):
            pltpu.make_async_copy(hs_c.at[pl.ds(0, EB), :],
                                  gbuf.at[p], semg[p]).wait()

        def scatter_fire(p, s4):
            pltpu.async_copy(gbuf.at[p], acc.at[colg.at[s4]], semsc[p],
                             add=True)

        def scatter_wait(p):
            pltpu.make_async_copy(hs_c.at[pl.ds(0, EB), :],
                                  gbuf.at[p], semsc[p]).wait()

        def full_iter(g, k):
            p = k % 2
            scatter_wait(p)
            gather_fire(k, p)
            gather_wait(p ^ 1)
            scatter_fire(p ^ 1, (k - 1) % 4)
            stage(jnp.minimum(g + 2, ghi - 1), (k + 2) % 4)

        for k0 in range(4):
            @pl.when(glo % 4 == k0)
            def _(k0=k0):
                k1 = (k0 + 1) % 4
                stage(glo, k0)
                stage(glo + 1, k1)
                gather_fire(k0, k0 % 2)
                stage(jnp.minimum(glo + 2, ghi - 1), (k0 + 2) % 4)
                gather_fire(k1, k1 % 2)
                gather_wait(k0 % 2)
                scatter_fire(k0 % 2, k0)
                stage(jnp.minimum(glo + 3, ghi - 1), (k1 + 2) % 4)

        def body(g, carry):
            for k in range(4):
                @pl.when(g % 4 == k)
                def _(k=k):
                    full_iter(g, k)
            return carry

        lax.fori_loop(glo + 2, ghi, body, None)

        for kl in range(4):
            @pl.when((ghi - 1) % 4 == kl)
            def _(kl=kl):
                pl_ = kl % 2
                gather_wait(pl_)
                scatter_fire(pl_, kl)
                scatter_wait(pl_ ^ 1)
                scatter_wait(pl_)
                stage_wait((kl + 1) % 4)
                stage_wait((kl + 2) % 4)

    def writeback(chunk):
        for k in range(RPT // RPB):
            off = tid * RPT + k * RPB
            pltpu.sync_copy(acc.at[pl.ds(off, RPB), :], bb)
            pltpu.sync_copy(bb, agg_out.at[chunk, pl.ds(off, RPB), :])

    for p in range(2):
        def zfill(i, carry):
            for j in range(CW // 16):
                bb[i, pl.ds(j * 16, 16)] = z16
            return carry

        lax.fori_loop(0, RPB, zfill, None)
        for k in range(RPT // RPB):
            pltpu.sync_copy(bb, acc.at[pl.ds(tid * RPT + k * RPB, RPB), :])
        plsc.subcore_barrier()

        @pl.when(core == 0)
        def _():
            edge_loop(2 * p)

        @pl.when(core == 1)
        def _():
            edge_loop(2 * p + 1)

        plsc.subcore_barrier()

        @pl.when(core == 0)
        def _():
            writeback(2 * p)

        @pl.when(core == 1)
        def _():
            writeback(2 * p + 1)

        plsc.subcore_barrier()


def _agg(row2d, col2d, hs4):
    k = pl.kernel(
        _agg_body,
        out_type=jax.ShapeDtypeStruct((NCHUNK, NP, CW), F32),
        mesh=_sc_mesh(),
        scratch_types=[
            pltpu.VMEM_SHARED((NP, CW), F32),
            pltpu.VMEM((4, EB), jnp.int32),
            pltpu.VMEM((4, EB), jnp.int32),
            pltpu.VMEM((2, EB, CW), F32),
            pltpu.VMEM((RPB, CW), F32),
            pltpu.SemaphoreType.DMA,
            pltpu.SemaphoreType.DMA,
            pltpu.SemaphoreType.DMA,
            pltpu.SemaphoreType.DMA,
            pltpu.SemaphoreType.DMA,
            pltpu.SemaphoreType.DMA,
            pltpu.SemaphoreType.DMA,
            pltpu.SemaphoreType.DMA,
        ],
    )
    return k(row2d, col2d, hs4)


RB = 1000
GRID = N // RB


def _prep_body(cnt_ref, dis_ref):
    c = cnt_ref[...]
    ones = jnp.ones((NC, 1), F32)
    deg = lax.dot_general(c, ones, (((0,), (0,)), ((), ())),
                          precision=HIGHEST) + 1.0
    dis_ref[...] = lax.rsqrt(deg)[:N]


def _prep(cnt):
    return pl.pallas_call(
        _prep_body,
        out_shape=jax.ShapeDtypeStruct((N, 1), F32),
    )(cnt)


def _dense1_body(x_ref, w_ref, dis_ref, hs_ref):
    y = lax.dot_general(x_ref[...], w_ref[...], (((1,), (0,)), ((), ())),
                        precision=HIGHEST)
    dis = dis_ref[...]
    for c in range(NCHUNK):
        hs_ref[c] = y[:, c * CW:(c + 1) * CW] * dis


def _dense1(x, w1, dis2d):
    return pl.pallas_call(
        _dense1_body,
        grid=(GRID,),
        in_specs=[
            pl.BlockSpec((RB, D), lambda i: (i, 0)),
            pl.BlockSpec((D, H), lambda i: (0, 0)),
            pl.BlockSpec((RB, 1), lambda i: (i, 0)),
        ],
        out_specs=pl.BlockSpec((NCHUNK, RB, CW), lambda i: (0, i, 0)),
        out_shape=jax.ShapeDtypeStruct((NCHUNK, N, CW), F32),
    )(x, w1, dis2d)


def _leaky(t):
    return jnp.where(t >= 0, t, 0.01 * t)


def _combine_h(agg_ref, hs_ref, dis_ref, b_ref):
    dis = dis_ref[...]
    bfull = b_ref[...]
    parts = []
    for c in range(NCHUNK):
        t = dis * (agg_ref[c] + hs_ref[c]) + bfull[c * CW:(c + 1) * CW][None, :]
        parts.append(_leaky(t))
    return jnp.concatenate(parts, axis=1)


def _dense2_body(agg_ref, hs_ref, dis_ref, b_ref, w_ref, out_ref):
    h = _combine_h(agg_ref, hs_ref, dis_ref, b_ref)
    y = lax.dot_general(h, w_ref[...], (((1,), (0,)), ((), ())),
                        precision=HIGHEST)
    dis = dis_ref[...]
    for c in range(NCHUNK):
        out_ref[c] = y[:, c * CW:(c + 1) * CW] * dis


def _dense2(agg, hs, dis2d, b_prev, w):
    return pl.pallas_call(
        _dense2_body,
        grid=(GRID,),
        in_specs=[
            pl.BlockSpec((NCHUNK, RB, CW), lambda i: (0, i, 0)),
            pl.BlockSpec((NCHUNK, RB, CW), lambda i: (0, i, 0)),
            pl.BlockSpec((RB, 1), lambda i: (i, 0)),
            pl.BlockSpec((H,), lambda i: (0,)),
            pl.BlockSpec((H, H), lambda i: (0, 0)),
        ],
        out_specs=pl.BlockSpec((NCHUNK, RB, CW), lambda i: (0, i, 0)),
        out_shape=jax.ShapeDtypeStruct((NCHUNK, N, CW), F32),
    )(agg, hs, dis2d, b_prev, w)


def _attn_a_body(agg_ref, hs_ref, dis_ref, b_ref, wv_ref, bv_ref, wu_ref,
                 bu_ref, wa_ref, ba_ref, h_ref, a_ref):
    h = _combine_h(agg_ref, hs_ref, dis_ref, b_ref)
    h_ref[...] = h
    av = jnp.tanh(lax.dot_general(h, wv_ref[...], (((1,), (0,)), ((), ())),
                                  precision=HIGHEST) + bv_ref[...][None, :])
    u = lax.dot_general(h, wu_ref[...], (((1,), (0,)), ((), ())),
                        precision=HIGHEST) + bu_ref[...][None, :]
    au = 1.0 / (1.0 + jnp.exp(-u))
    g = av * au
    a_ref[...] = lax.dot_general(g, wa_ref[...], (((1,), (0,)), ((), ())),
                                 precision=HIGHEST) + ba_ref[0, 0]


def _attn_a(agg, hs, dis2d, b3, wv, bv, wu, bu, wa, ba2):
    return pl.pallas_call(
        _attn_a_body,
        grid=(GRID,),
        in_specs=[
            pl.BlockSpec((NCHUNK, RB, CW), lambda i: (0, i, 0)),
            pl.BlockSpec((NCHUNK, RB, CW), lambda i: (0, i, 0)),
            pl.BlockSpec((RB, 1), lambda i: (i, 0)),
            pl.BlockSpec((H,), lambda i: (0,)),
            pl.BlockSpec((H, CW), lambda i: (0, 0)),
            pl.BlockSpec((CW,), lambda i: (0,)),
            pl.BlockSpec((H, CW), lambda i: (0, 0)),
            pl.BlockSpec((CW,), lambda i: (0,)),
            pl.BlockSpec((CW, 1), lambda i: (0, 0)),
            pl.BlockSpec((1, 1), lambda i: (0, 0)),
        ],
        out_specs=[
            pl.BlockSpec((RB, H), lambda i: (i, 0)),
            pl.BlockSpec((RB, 1), lambda i: (i, 0)),
        ],
        out_shape=[
            jax.ShapeDtypeStruct((N, H), F32),
            jax.ShapeDtypeStruct((N, 1), F32),
        ],
    )(agg, hs, dis2d, b3, wv, bv, wu, bu, wa, ba2)


def _pool_body(h_ref, a_in_ref, batch_ref, wc_ref, bc_ref, a_ref, out_ref):
    bidx = batch_ref[...]
    lanes = lax.broadcasted_iota(jnp.int32, (1, B), 1)
    maskb = bidx == lanes
    mask = maskb.astype(F32)
    av = a_in_ref[...]
    neg = jnp.float32(-1e30)
    m_seg = jnp.max(jnp.where(maskb, av, neg), axis=0, keepdims=True)
    m_node = jnp.max(jnp.where(maskb, m_seg, neg), axis=1, keepdims=True)
    e = jnp.exp(av - m_node)
    s_seg = jnp.sum(mask * e, axis=0, keepdims=True)
    s_node = jnp.sum(mask * s_seg, axis=1, keepdims=True)
    a = e / (s_node + 1e-16)
    a_ref[...] = a
    w = mask * a
    z = lax.dot_general(w, h_ref[...], (((0,), (0,)), ((), ())),
                        precision=lax.Precision.HIGHEST)
    out_ref[...] = lax.dot_general(z, wc_ref[...], (((1,), (0,)), ((), ())),
                                   precision=HIGHEST) + bc_ref[0, 0]


def _pool(h3, a_node, batch2d, wc, bc2):
    return pl.pallas_call(
        _pool_body,
        out_shape=[
            jax.ShapeDtypeStruct((N, 1), F32),
            jax.ShapeDtypeStruct((B, 1), F32),
        ],
    )(h3, a_node, batch2d, wc, bc2)


def kernel(x, edge_index, batch, W1, b1, W2, b2, W3, b3, WV, bV, WU, bU,
           Wa, ba, Wc, bc):
    row2d = edge_index[0].reshape(NB, EB)
    col2d = edge_index[1].reshape(NB, EB)
    batch2d = batch.reshape(N, 1)
    ba2 = ba.reshape(1, 1)
    bc2 = bc.reshape(1, 1)

    cnt = _hist(col2d)
    dis2d = _prep(cnt)

    hs1 = _dense1(x, W1, dis2d)
    agg1 = _agg(row2d, col2d, hs1)
    hs2 = _dense2(agg1, hs1, dis2d, b1, W2)
    agg2 = _agg(row2d, col2d, hs2)
    hs3 = _dense2(agg2, hs2, dis2d, b2, W3)
    agg3 = _agg(row2d, col2d, hs3)
    h3, a_node = _attn_a(agg3, hs3, dis2d, b3, WV, bV, WU, bU, Wa, ba2)
    a2d, outv = _pool(h3, a_node, batch2d, Wc, bc2)

    return outv.reshape(B), a2d.reshape(N), h3
